# baseline (speedup 1.0000x reference)
import functools

import jax
import jax.numpy as jnp
from jax import lax
from jax.experimental import pallas as pl
from jax.experimental.pallas import tpu as pltpu
from jax.experimental.pallas import tpu_sc as plsc

VOCAB = 1000000
D = 128
C = 2
B = 4096
S = 200

NC = 2
NS = 16
NW = NC * NS
RW = B // NW
NG = RW // 16
NT = RW // 3 + 1
CH = (56, 48, 48, 48)
CO = (0, 56, 104, 152)
NV = D // 16

_mesh = plsc.VectorSubcoreMesh(core_axis_name="c", subcore_axis_name="s")


@functools.partial(
    pl.kernel,
    out_type=jax.ShapeDtypeStruct((B * C,), jnp.float32),
    mesh=_mesh,
    compiler_params=pltpu.CompilerParams(needs_layout_passes=False, use_tc_tiling_on_sc=False),
    scratch_types=[
        pltpu.VMEM((RW, S), jnp.int32),
        pltpu.VMEM((S, D), jnp.float32),
        pltpu.VMEM((S, D), jnp.float32),
        pltpu.VMEM((S, D), jnp.float32),
        pltpu.VMEM((C, D), jnp.float32),
        pltpu.VMEM((16,), jnp.float32),
        pltpu.VMEM((256,), jnp.float32),
        pltpu.VMEM((256,), jnp.float32),
        pltpu.VMEM((32,), jnp.float32),
        pltpu.VMEM((RW * C,), jnp.float32),
        pltpu.SemaphoreType.DMA,
        pltpu.SemaphoreType.DMA,
        pltpu.SemaphoreType.DMA,
    ],
)
def _fasttext_sc(x_hbm, emb_hbm, w_hbm, b_hbm, out_hbm,
                 idx_all, rows0, rows1, rows2, w_v, b_v, pbuf0, pbuf1,
                 itl, out_v, sem0, sem1, sem2):
    wid = lax.axis_index("s") * NC + lax.axis_index("c")
    base = wid * RW
    pltpu.sync_copy(x_hbm.at[pl.ds(base, RW)], idx_all)
    pltpu.sync_copy(w_hbm, w_v)
    pltpu.sync_copy(b_hbm, b_v.at[pl.ds(0, C)])
    inv_s = jnp.float32(1.0 / S)
    bvec = b_v[...]
    lane = lax.iota(jnp.int32, 16)
    ilv0 = (lane // 2) + (lane % 2) * 16
    ilv1 = ilv0 + 8
    w_regs = [[w_v[c, pl.ds(j * 16, 16)] for j in range(NV)] for c in range(C)]

    bufs = (rows0, rows1, rows2)
    sems = (sem0, sem1, sem2)

    def fire(i, rows_buf, sem):
        r = jnp.minimum(i, RW - 1)
        for n, o in zip(CH, CO):
            pltpu.async_copy(
                emb_hbm.at[idx_all.at[r, pl.ds(o, n)]],
                rows_buf.at[pl.ds(o, n)], sem)

    def drain(rows_buf, sem):
        for n, o in zip(CH, CO):
            pltpu.make_async_copy(
                emb_hbm.at[idx_all.at[0, pl.ds(0, n)]],
                rows_buf.at[pl.ds(o, n)], sem).wait()

    def reduce_project(rows_buf, i):
        def red(s, acc):
            return tuple(
                acc[j] + rows_buf[s, pl.ds(j * 16, 16)] for j in range(NV))

        acc = lax.fori_loop(
            0, S, red,
            tuple(jnp.zeros((16,), jnp.float32) for _ in range(NV)),
            unroll=8)
        k = i % 16
        for c, pb in ((0, pbuf0), (1, pbuf1)):
            pv = acc[0] * w_regs[c][0]
            for j in range(1, NV):
                pv = pv + acc[j] * w_regs[c][j]
            pb[pl.ds(k * 16, 16)] = pv

        @pl.when(k == 15)
        def _flush():
            for c, pb in ((0, pbuf0), (1, pbuf1)):
                tot = jnp.zeros((16,), jnp.float32)
                for l in range(16):
                    tot = tot + plsc.load_gather(pb, [lane * 16 + l])
                itl[pl.ds(c * 16, 16)] = tot * inv_s + bvec[c]
            g32 = (i // 16) * 32
            out_v[pl.ds(g32, 16)] = plsc.load_gather(itl, [ilv0])
            out_v[pl.ds(g32 + 16, 16)] = plsc.load_gather(itl, [ilv1])

    fire(0, bufs[0], sems[0])
    fire(1, bufs[1], sems[1])

    def triple_body(t, carry):
        i0 = 3 * t
        for j in range(3):
            fire(i0 + j + 2, bufs[(j + 2) % 3], sems[(j + 2) % 3])
            drain(bufs[j], sems[j])
            reduce_project(bufs[j], i0 + j)
        return carry

    lax.fori_loop(0, NT, triple_body, 0)
    drain(bufs[0], sems[0])
    drain(bufs[1], sems[1])
    pltpu.sync_copy(out_v, out_hbm.at[pl.ds(base * C, RW * C)])


def kernel(x, emb, W, b):
    out = _fasttext_sc(x.astype(jnp.int32), emb, W, b.astype(jnp.float32))
    return out.reshape(B, C)


# speedup vs baseline: 1.0017x; 1.0017x over previous
import functools

import jax
import jax.numpy as jnp
from jax import lax
from jax.experimental import pallas as pl
from jax.experimental.pallas import tpu as pltpu
from jax.experimental.pallas import tpu_sc as plsc

VOCAB = 1000000
D = 128
C = 2
B = 4096
S = 200

NC = 2
NS = 16
NW = NC * NS
RW = B // NW
NG = RW // 16
NT = RW // 3 + 1
CH = (56, 48, 48, 48)
CO = (0, 56, 104, 152)
NV = D // 16

_mesh = plsc.VectorSubcoreMesh(core_axis_name="c", subcore_axis_name="s")


@functools.partial(
    pl.kernel,
    out_type=jax.ShapeDtypeStruct((B * C,), jnp.float32),
    mesh=_mesh,
    compiler_params=pltpu.CompilerParams(needs_layout_passes=False),
    scratch_types=[
        pltpu.VMEM((RW * S,), jnp.int32),
        pltpu.VMEM((S, D), jnp.float32),
        pltpu.VMEM((S, D), jnp.float32),
        pltpu.VMEM((S, D), jnp.float32),
        pltpu.VMEM((C, D), jnp.float32),
        pltpu.VMEM((16,), jnp.float32),
        pltpu.VMEM((256,), jnp.float32),
        pltpu.VMEM((256,), jnp.float32),
        pltpu.VMEM((32,), jnp.float32),
        pltpu.VMEM((RW * C,), jnp.float32),
        pltpu.SemaphoreType.DMA,
        pltpu.SemaphoreType.DMA,
        pltpu.SemaphoreType.DMA,
    ],
)
def _fasttext_sc(x_hbm, emb_hbm, w_hbm, b_hbm, out_hbm,
                 idx_all, rows0, rows1, rows2, w_v, b_v, pbuf0, pbuf1,
                 itl, out_v, sem0, sem1, sem2):
    wid = lax.axis_index("s") * NC + lax.axis_index("c")
    base = wid * RW
    pltpu.sync_copy(x_hbm.at[pl.ds(base * S, RW * S)], idx_all)
    pltpu.sync_copy(w_hbm, w_v)
    pltpu.sync_copy(b_hbm, b_v.at[pl.ds(0, C)])
    inv_s = jnp.float32(1.0 / S)
    bvec = b_v[...]
    lane = lax.iota(jnp.int32, 16)
    ilv0 = (lane // 2) + (lane % 2) * 16
    ilv1 = ilv0 + 8
    w_regs = [[w_v[c, pl.ds(j * 16, 16)] for j in range(NV)] for c in range(C)]

    bufs = (rows0, rows1, rows2)
    sems = (sem0, sem1, sem2)

    def fire(i, rows_buf, sem):
        off = jnp.minimum(i, RW - 1) * S
        for n, o in zip(CH, CO):
            pltpu.async_copy(
                emb_hbm.at[idx_all.at[pl.ds(off + o, n)]],
                rows_buf.at[pl.ds(o, n)], sem)

    def drain(rows_buf, sem):
        for n, o in zip(CH, CO):
            pltpu.make_async_copy(
                emb_hbm.at[idx_all.at[pl.ds(0, n)]],
                rows_buf.at[pl.ds(o, n)], sem).wait()

    def reduce_project(rows_buf, i):
        def red(s, acc):
            return tuple(
                acc[j] + rows_buf[s, pl.ds(j * 16, 16)] for j in range(NV))

        acc = lax.fori_loop(
            0, S, red,
            tuple(jnp.zeros((16,), jnp.float32) for _ in range(NV)),
            unroll=8)
        k = i % 16
        for c, pb in ((0, pbuf0), (1, pbuf1)):
            pv = acc[0] * w_regs[c][0]
            for j in range(1, NV):
                pv = pv + acc[j] * w_regs[c][j]
            pb[pl.ds(k * 16, 16)] = pv

        @pl.when(k == 15)
        def _flush():
            for c, pb in ((0, pbuf0), (1, pbuf1)):
                tot = jnp.zeros((16,), jnp.float32)
                for l in range(16):
                    tot = tot + plsc.load_gather(pb, [lane * 16 + l])
                itl[pl.ds(c * 16, 16)] = tot * inv_s + bvec[c]
            g32 = (i // 16) * 32
            out_v[pl.ds(g32, 16)] = plsc.load_gather(itl, [ilv0])
            out_v[pl.ds(g32 + 16, 16)] = plsc.load_gather(itl, [ilv1])

    fire(0, bufs[0], sems[0])
    fire(1, bufs[1], sems[1])

    def triple_body(t, carry):
        i0 = 3 * t
        for j in range(3):
            fire(i0 + j + 2, bufs[(j + 2) % 3], sems[(j + 2) % 3])
            drain(bufs[j], sems[j])
            reduce_project(bufs[j], i0 + j)
        return carry

    lax.fori_loop(0, NT, triple_body, 0)
    drain(bufs[0], sems[0])
    drain(bufs[1], sems[1])
    pltpu.sync_copy(out_v, out_hbm.at[pl.ds(base * C, RW * C)])


def kernel(x, emb, W, b):
    out = _fasttext_sc(x.astype(jnp.int32).reshape(-1), emb, W,
                       b.astype(jnp.float32))
    return out.reshape(B, C)
